# Initial kernel scaffold; baseline (speedup 1.0000x reference)
import jax
import jax.numpy as jnp
from jax.experimental import pallas as pl


def kernel(obs_ids, table, gamma, beta, W1, b1, W2, b2, Wv1, bv1, Wv2, bv2):
    raise NotImplementedError("write your pallas kernel here")


# speedup vs baseline: 1.5803x; 1.5803x over previous
import functools

import jax
import jax.numpy as jnp
from jax import lax
from jax.experimental import pallas as pl
from jax.experimental.pallas import tpu as pltpu
from jax.experimental.pallas import tpu_sc as plsc

_VOCAB = 100000
_D = 64
_B = 4096
_L = 200

_NC = 2
_NS = 16
_NW = _NC * _NS
_BPW = _B // _NW
_CHUNK = 100
_NCHUNK = _L // _CHUNK
_ROWS = _BPW * _NCHUNK

_VB = 1024
_NV = (_VOCAB + _VB - 1) // _VB


def _accum4(rows_ref, acc):
    def body(t, a):
        return tuple(a[q] + rows_ref[t, pl.ds(16 * q, 16)] for q in range(4))
    return plsc.parallel_loop(0, _CHUNK, unroll=10, carry=acc)(body)


def _pool_body(ids_hbm, table_hbm, out_hbm, idx_v, rows0, rows1, out_v,
               sem0, sem1):
    c = lax.axis_index("c")
    s = lax.axis_index("s")
    wid = s * _NC + c
    pltpu.sync_copy(ids_hbm.at[pl.ds(wid * _ROWS, _ROWS)], idx_v)
    pltpu.async_copy(table_hbm.at[idx_v.at[0]], rows0, sem0)

    def row_body(r, carry):
        k0 = 2 * r
        cp1 = pltpu.async_copy(table_hbm.at[idx_v.at[k0 + 1]], rows1, sem1)
        pltpu.make_async_copy(table_hbm.at[idx_v.at[k0]], rows0, sem0).wait()
        zero = jnp.zeros((16,), jnp.float32)
        acc = _accum4(rows0, (zero, zero, zero, zero))

        @pl.when(r < _BPW - 1)
        def _start_next():
            pltpu.async_copy(table_hbm.at[idx_v.at[k0 + 2]], rows0, sem0)

        cp1.wait()
        acc = _accum4(rows1, acc)
        for q in range(4):
            out_v[r, pl.ds(16 * q, 16)] = acc[q]
        return carry

    lax.fori_loop(0, _BPW, row_body, 0)
    pltpu.sync_copy(out_v, out_hbm.at[pl.ds(wid * _BPW, _BPW)])


@functools.cache
def _make_pool():
    return pl.kernel(
        _pool_body,
        out_type=jax.ShapeDtypeStruct((_B, _D), jnp.float32),
        mesh=plsc.VectorSubcoreMesh(core_axis_name="c", subcore_axis_name="s"),
        scratch_types=[
            pltpu.VMEM((_ROWS, _CHUNK), jnp.int32),
            pltpu.VMEM((_CHUNK, _D), jnp.float32),
            pltpu.VMEM((_CHUNK, _D), jnp.float32),
            pltpu.VMEM((_BPW, _D), jnp.float32),
            pltpu.SemaphoreType.DMA,
            pltpu.SemaphoreType.DMA,
        ],
        compiler_params=pltpu.CompilerParams(use_tc_tiling_on_sc=False),
    )


def _head_body(pooled_ref, gamma_ref, beta_ref, W1_ref, b1_ref, Wv1_ref,
               bv1_ref, Wv2_ref, bv2_ref, W2_ref, b2_ref,
               logits_ref, value_ref, h_scr):
    v = pl.program_id(0)

    @pl.when(v == 0)
    def _small_stage():
        x = pooled_ref[...] * (1.0 / _L)
        mu = jnp.mean(x, axis=-1, keepdims=True)
        xc = x - mu
        var = jnp.mean(xc * xc, axis=-1, keepdims=True)
        xn = xc * lax.rsqrt(var + 1e-5) * gamma_ref[...] + beta_ref[...]
        h = jnp.tanh(
            jnp.dot(xn, W1_ref[...], preferred_element_type=jnp.float32)
            + b1_ref[...])
        h_scr[...] = h
        hv = jnp.tanh(
            jnp.dot(xn, Wv1_ref[...], preferred_element_type=jnp.float32)
            + bv1_ref[...])
        value_ref[...] = (
            jnp.dot(hv, Wv2_ref[...], preferred_element_type=jnp.float32)
            + bv2_ref[...])

    logits_ref[...] = (
        jnp.dot(h_scr[...], W2_ref[...], preferred_element_type=jnp.float32)
        + b2_ref[...])


@functools.cache
def _make_heads():
    full = lambda shape: pl.BlockSpec(shape, lambda v: (0,) * len(shape))
    return pl.pallas_call(
        _head_body,
        grid=(_NV,),
        in_specs=[
            full((_B, _D)),
            full((1, _D)),
            full((1, _D)),
            full((_D, _D)),
            full((1, _D)),
            full((_D, _D)),
            full((1, _D)),
            full((_D, 1)),
            full((1, 1)),
            pl.BlockSpec((_D, _VB), lambda v: (0, v)),
            pl.BlockSpec((1, _VB), lambda v: (0, v)),
        ],
        out_specs=[
            pl.BlockSpec((_B, _VB), lambda v: (0, v)),
            pl.BlockSpec((_B, 1), lambda v: (0, 0)),
        ],
        out_shape=[
            jax.ShapeDtypeStruct((_B, _VOCAB), jnp.float32),
            jax.ShapeDtypeStruct((_B, 1), jnp.float32),
        ],
        scratch_shapes=[pltpu.VMEM((_B, _D), jnp.float32)],
    )


def kernel(obs_ids, table, gamma, beta, W1, b1, W2, b2, Wv1, bv1, Wv2, bv2):
    ids2 = obs_ids.reshape(_B * _NCHUNK, _CHUNK).astype(jnp.int32)
    pooled = _make_pool()(ids2, table)
    logits, value = _make_heads()(
        pooled,
        gamma.reshape(1, _D), beta.reshape(1, _D),
        W1, b1.reshape(1, _D),
        Wv1, bv1.reshape(1, _D),
        Wv2, bv2.reshape(1, 1),
        W2, b2.reshape(1, _VOCAB),
    )
    return logits, value.reshape(_B)
